# baseline (speedup 1.0000x reference)
import functools

import jax
import jax.numpy as jnp
from jax import lax
from jax.experimental import pallas as pl
from jax.experimental.pallas import tpu as pltpu
from jax.experimental.pallas import tpu_sc as plsc

BATCH = 16384
D = 64
L = 16
CW = 256
NCHUNK = 999936 // CW
TAIL0 = NCHUNK * CW
STAGE = 128
FLUSH_AT = STAGE - L
DUMMY = BATCH
NVREG = BATCH // L


@functools.cache
def _build_sc():
    info = plsc.get_sparse_core_info()
    NC = info.num_cores
    NW = NC * info.num_subcores
    base_nt = NCHUNK // NW
    rem = NCHUNK - base_nt * NW
    mesh = plsc.VectorSubcoreMesh(core_axis_name="c", subcore_axis_name="s")

    @functools.partial(
        pl.kernel,
        mesh=mesh,
        out_type=(jax.ShapeDtypeStruct((BATCH + L, 128), jnp.float32),
                  jax.ShapeDtypeStruct((BATCH + L, 128), jnp.float32)),
        compiler_params=pltpu.CompilerParams(
            needs_layout_passes=False, use_tc_tiling_on_sc=True),
        scratch_types=(
            [pltpu.VMEM((BATCH,), jnp.int32)]
            + [pltpu.VMEM((BATCH + L,), jnp.int32)] * 2
            + [pltpu.VMEM((48,), jnp.int32)] * 2
            + [pltpu.VMEM((D, CW), jnp.float32)] * 2
            + [pltpu.VMEM((STAGE, 128), jnp.float32)]
            + [pltpu.VMEM((STAGE,), jnp.int32)]
            + [pltpu.SemaphoreType.DMA] * 3
        ),
    )
    def scan(u_ids_hbm, i_ids_hbm, ut_hbm, it_hbm, ue_hbm, ie_hbm,
             idsv, listb, listid, pendb, pendj, buf0, buf1, stage, bidx,
             sem0, sem1, semf):
        bufs = (buf0, buf1)
        sems = (sem0, sem1)
        wid = lax.axis_index("s") * NC + lax.axis_index("c")
        nt = base_nt + jnp.where(wid < rem, 1, 0)
        iota = lax.iota(jnp.int32, L)

        def reset_bidx():
            for j in range(STAGE // L):
                bidx[pl.ds(j * L, L)] = jnp.full((L,), DUMMY, jnp.int32)

        def one_table(ids_hbm, tab, out_hbm):
            pltpu.sync_copy(ids_hbm, idsv)

            def sel(i, cnt):
                v = idsv[pl.ds(i * L, L)]
                m = jnp.bitwise_and(
                    lax.shift_right_logical(v, 8), NW - 1) == wid
                plsc.store_compressed(
                    listb.at[pl.ds(cnt, L)], i * L + iota, mask=m)
                plsc.store_compressed(
                    listid.at[pl.ds(cnt, L)], v, mask=m)
                return cnt + jnp.sum(m.astype(jnp.int32))

            cnt = lax.fori_loop(0, NVREG, sel, jnp.int32(0))
            nq = (cnt + (L - 1)) // L
            reset_bidx()

            def fire(t, s):
                col0 = pl.multiple_of((t * NW + wid) * CW, 128)
                pltpu.async_copy(tab.at[:, pl.ds(col0, CW)], bufs[s], sems[s])

            def drain(s):
                pltpu.make_async_copy(
                    tab.at[:, pl.ds(0, CW)], bufs[s], sems[s]).wait()

            def flush():
                pltpu.async_copy(stage, out_hbm.at[bidx], semf).wait()
                reset_bidx()

            @pl.when(nt > 0)
            def _():
                fire(jnp.int32(0), 0)

            @pl.when(nt > 1)
            def _():
                fire(jnp.int32(1), 1)

            def consume(s, npend, nslots, take):
                bvec = pendb[pl.ds(0, L)]
                jvec = jnp.bitwise_and(pendj[pl.ds(0, L)], CW - 1)
                m = iota < take
                slots = nslots + plsc.cumsum(m.astype(jnp.int32)) - 1
                for d in range(D):
                    vals = plsc.load_gather(
                        bufs[s], [jnp.full((L,), d, jnp.int32), jvec])
                    plsc.store_scatter(
                        stage, [slots, jnp.full((L,), d, jnp.int32)],
                        vals, mask=m)
                plsc.store_scatter(bidx, [slots], bvec, mask=m)
                pendb[pl.ds(0, L)] = pendb[pl.ds(L, L)]
                pendj[pl.ds(0, L)] = pendj[pl.ds(L, L)]
                nslots2 = nslots + take

                @pl.when(nslots2 >= FLUSH_AT)
                def _():
                    flush()

                return (npend - take,
                        jnp.where(nslots2 >= FLUSH_AT, 0, nslots2))

            def round_body(r, carry):
                for s in range(2):
                    t = r * 2 + s

                    def scanq(q, car):
                        npend, nslots = car
                        vb = listb[pl.ds(q * L, L)]
                        vid = listid[pl.ds(q * L, L)]
                        valid = (q * L + iota) < cnt
                        m = (lax.shift_right_logical(vid, 8)
                             == (t * NW + wid)) & valid
                        nm = jnp.sum(m.astype(jnp.int32))

                        def have(car2):
                            npend2, nslots2 = car2
                            plsc.store_compressed(
                                pendb.at[pl.ds(npend2, L)], vb, mask=m)
                            plsc.store_compressed(
                                pendj.at[pl.ds(npend2, L)], vid, mask=m)
                            npend3 = npend2 + nm
                            return lax.cond(
                                npend3 >= L,
                                lambda c: consume(s, c[0], c[1], jnp.int32(L)),
                                lambda c: c,
                                (npend3, nslots2))

                        return lax.cond(nm > 0, have, lambda c: c, car)

                    def do_chunk(car):
                        drain(s)
                        npend, nslots = lax.fori_loop(0, nq, scanq, car)
                        npend, nslots = lax.cond(
                            npend > 0,
                            lambda c: consume(s, c[0], c[1], c[0]),
                            lambda c: c,
                            (npend, nslots))

                        @pl.when(t + 2 < nt)
                        def _():
                            fire(t + 2, s)

                        return (npend, nslots)

                    carry = lax.cond(t < nt, do_chunk, lambda c: c, carry)
                return carry

            nrounds = (base_nt + 1 + 1) // 2
            lax.fori_loop(0, nrounds, round_body,
                          (jnp.int32(0), jnp.int32(0)))
            flush()

        one_table(u_ids_hbm, ut_hbm, ue_hbm)
        one_table(i_ids_hbm, it_hbm, ie_hbm)

    return scan


@functools.cache
def _build_tc():
    BLK = 2048
    grid = BATCH // BLK

    def body(ue_ref, ie_ref, uid_ref, iid_ref, utail_ref, itail_ref, out_ref):
        uid = uid_ref[...]
        iid = iid_ref[...]
        io64 = lax.broadcasted_iota(jnp.int32, (BLK, D), 1)

        def patch(rows, ids, tail_ref):
            flag = ids >= TAIL0
            oh = (io64 == (ids - TAIL0)).astype(jnp.float32)
            trows = jax.lax.dot_general(
                oh, tail_ref[...], (((1,), (0,)), ((), ())),
                precision=jax.lax.Precision.HIGHEST,
                preferred_element_type=jnp.float32)
            return jnp.where(flag, trows, rows)

        ue = patch(ue_ref[:, :D], uid, utail_ref)
        ie = patch(ie_ref[:, :D], iid, itail_ref)
        out_ref[...] = jnp.sum(ue * ie, axis=1)

    return pl.pallas_call(
        body,
        grid=(grid,),
        in_specs=[
            pl.BlockSpec((BLK, 128), lambda i: (i, 0)),
            pl.BlockSpec((BLK, 128), lambda i: (i, 0)),
            pl.BlockSpec((BLK, 1), lambda i: (i, 0)),
            pl.BlockSpec((BLK, 1), lambda i: (i, 0)),
            pl.BlockSpec((D, D), lambda i: (0, 0)),
            pl.BlockSpec((D, D), lambda i: (0, 0)),
        ],
        out_specs=pl.BlockSpec((BLK,), lambda i: (i,)),
        out_shape=jax.ShapeDtypeStruct((BATCH,), jnp.float32),
    )


def kernel(u_ids, i_ids, user_table, item_table):
    uid = u_ids.astype(jnp.int32)
    iid = i_ids.astype(jnp.int32)
    ue, ie = _build_sc()(uid, iid, user_table.T, item_table.T)
    return _build_tc()(ue, ie, uid[:, None], iid[:, None],
                       user_table[TAIL0:], item_table[TAIL0:])
